# baseline (speedup 1.0000x reference)
import jax
import jax.numpy as jnp
from jax import lax
from jax.experimental import pallas as pl
from jax.experimental.pallas import tpu as pltpu
from jax.experimental.pallas import tpu_sc as plsc

B = 4096
N = 64
NC, NS = 2, 16
NW = NC * NS
BPW = B // NW
GRP = BPW // 16
NG = 8
RPG = N // NG


def _sc_body(xf_hbm, idx_hbm, maskb_hbm, act_hbm, bsel_hbm, out_hbm, p_hbm,
             idx_v, vals_v, maskb_v, act_v, bsel_v, sat_v, out_v, sems):
    _ZERO = jnp.zeros((16,), jnp.float32)
    _ONE = jnp.ones((16,), jnp.float32)
    _EPS = jnp.full((16,), 1e-20, jnp.float32)
    _THR = jnp.full((16,), 0.8, jnp.float32)
    wid = lax.axis_index("s") * NC + lax.axis_index("c")

    pltpu.sync_copy(idx_hbm.at[wid], idx_v)

    CHUNK = RPG * BPW
    handles = [
        pltpu.async_copy(
            xf_hbm.at[idx_v.at[pl.ds(g * CHUNK, CHUNK)]],
            vals_v.at[pl.ds(g * CHUNK, CHUNK)],
            sems[g],
        )
        for g in range(NG)
    ]

    pltpu.sync_copy(maskb_hbm, maskb_v)
    pltpu.sync_copy(act_hbm, act_v)
    pltpu.sync_copy(bsel_hbm, bsel_v)

    a = act_v[...]
    probs = a / (a + _EPS)

    def body_i(i, accs):
        mrow = maskb_v[i, :]
        out = []
        for g in range(GRP):
            v = vals_v[pl.ds(i * BPW + g * 16, 16)]
            predf = jnp.where(v > _THR, _ONE, _ZERO)
            out.append(accs[g] + jnp.abs(predf - mrow))
        return tuple(out)

    accs = tuple(_ZERO for _ in range(GRP))
    for g in range(NG):
        handles[g].wait()
        accs = lax.fori_loop(g * RPG, (g + 1) * RPG, body_i, accs)

    for g in range(GRP):
        sat_v[pl.ds(g * 16, 16)] = jnp.where(accs[g] == _ZERO, _ONE, _ZERO)

    for t in range(BPW // 2):
        sv = plsc.load_gather(sat_v, [bsel_v[t, :]])
        out_v[t, :] = sv * probs

    pltpu.sync_copy(out_v, out_hbm.at[pl.ds(wid * (BPW // 2), BPW // 2)])

    for r in range(GRP):
        out_v[r, :] = _ZERO
    pltpu.sync_copy(out_v.at[pl.ds(0, GRP)], p_hbm.at[pl.ds(wid * GRP, GRP)])


@jax.jit
def kernel(x, action, mask):
    xf = x.reshape(-1)
    w = jnp.arange(NW, dtype=jnp.int32)[:, None, None]
    i = jnp.arange(N, dtype=jnp.int32)[None, :, None]
    c = jnp.arange(BPW, dtype=jnp.int32)[None, None, :]
    idx = ((w * BPW + c) * (N * N) + i * (N + 1)).reshape(NW, N * BPW)
    maskb = jnp.broadcast_to(
        mask.astype(jnp.float32)[:, None], (N, 16))
    act2 = jnp.concatenate([action, action])
    bsel_all = (jnp.arange(16, dtype=jnp.int32)[None, :] // 8
                + 2 * jnp.arange(BPW // 2, dtype=jnp.int32)[:, None])

    mesh = plsc.VectorSubcoreMesh(
        core_axis_name="c", subcore_axis_name="s",
        num_cores=NC, num_subcores=NS)
    kfn = pl.kernel(
        _sc_body,
        out_type=(
            jax.ShapeDtypeStruct((B // 2, 16), jnp.float32),
            jax.ShapeDtypeStruct((B // 16, 16), jnp.float32),
        ),
        mesh=mesh,
        compiler_params=pltpu.CompilerParams(
            needs_layout_passes=False,
            skip_device_barrier=True,
            disable_bounds_checks=True,
            disable_semaphore_checks=True,
        ),
        scratch_types=[
            pltpu.VMEM((N * BPW,), jnp.int32),
            pltpu.VMEM((N * BPW,), jnp.float32),
            pltpu.VMEM((N, 16), jnp.float32),
            pltpu.VMEM((16,), jnp.float32),
            pltpu.VMEM((BPW // 2, 16), jnp.int32),
            pltpu.VMEM((BPW,), jnp.float32),
            pltpu.VMEM((BPW // 2, 16), jnp.float32),
            [pltpu.SemaphoreType.DMA] * NG,
        ],
    )
    out, pz = kfn(xf, idx, maskb, act2, bsel_all)
    return out.reshape(B, 8), pz.reshape(1, B)


# speedup vs baseline: 1.0060x; 1.0060x over previous
import jax
import jax.numpy as jnp
from jax import lax
from jax.experimental import pallas as pl
from jax.experimental.pallas import tpu as pltpu
from jax.experimental.pallas import tpu_sc as plsc

B = 4096
N = 64
NC, NS = 2, 16
NW = NC * NS
BPW = B // NW
GRP = BPW // 16
NG = 8
RPG = N // NG


def _sc_body(xf_hbm, idx_hbm, maskb_hbm, act_hbm, bsel_hbm, out_hbm, p_hbm,
             idx_v, vals_v, maskb_v, act_v, bsel_v, sat_v, out_v, sems):
    _ZERO = jnp.zeros((16,), jnp.float32)
    _ONE = jnp.ones((16,), jnp.float32)
    _EPS = jnp.full((16,), 1e-20, jnp.float32)
    _THR = jnp.full((16,), 0.8, jnp.float32)
    wid = lax.axis_index("s") * NC + lax.axis_index("c")

    pltpu.sync_copy(idx_hbm.at[wid], idx_v)

    CHUNK = RPG * BPW
    handles = [
        pltpu.async_copy(
            xf_hbm.at[idx_v.at[pl.ds(g * CHUNK, CHUNK)]],
            vals_v.at[pl.ds(g * CHUNK, CHUNK)],
            sems[g],
        )
        for g in range(NG)
    ]

    pltpu.sync_copy(maskb_hbm, maskb_v)
    pltpu.sync_copy(act_hbm, act_v)
    pltpu.sync_copy(bsel_hbm, bsel_v)

    a = act_v[...]
    probs = a / (a + _EPS)

    def body_i(i, accs):
        mrow = maskb_v[i, :]
        out = []
        for g in range(GRP):
            v = vals_v[pl.ds(i * BPW + g * 16, 16)]
            predf = jnp.where(v > _THR, _ONE, _ZERO)
            out.append(accs[g] + jnp.abs(predf - mrow))
        return tuple(out)

    accs = tuple(_ZERO for _ in range(GRP))
    for g in range(NG):
        handles[g].wait()
        accs = lax.fori_loop(g * RPG, (g + 1) * RPG, body_i, accs)

    for g in range(GRP):
        sat_v[pl.ds(g * 16, 16)] = jnp.where(accs[g] == _ZERO, _ONE, _ZERO)

    for t in range(BPW // 2):
        sv = plsc.load_gather(sat_v, [bsel_v[t, :]])
        out_v[t, :] = sv * probs

    pltpu.sync_copy(out_v, out_hbm.at[pl.ds(wid * (BPW // 2), BPW // 2)])

    for r in range(GRP):
        out_v[r, :] = _ZERO
    pltpu.sync_copy(out_v.at[pl.ds(0, GRP)], p_hbm.at[pl.ds(wid * GRP, GRP)])


@jax.jit
def kernel(x, action, mask):
    xf = x.reshape(-1)
    w = jnp.arange(NW, dtype=jnp.int32)[:, None, None]
    i = jnp.arange(N, dtype=jnp.int32)[None, :, None]
    c = jnp.arange(BPW, dtype=jnp.int32)[None, None, :]
    idx = ((w * BPW + c) * (N * N) + i * (N + 1)).reshape(NW, N * BPW)
    maskb = jnp.broadcast_to(
        mask.astype(jnp.float32)[:, None], (N, 16))
    act2 = jnp.concatenate([action, action])
    bsel_all = (jnp.arange(16, dtype=jnp.int32)[None, :] // 8
                + 2 * jnp.arange(BPW // 2, dtype=jnp.int32)[:, None])

    mesh = plsc.VectorSubcoreMesh(
        core_axis_name="c", subcore_axis_name="s",
        num_cores=NC, num_subcores=NS)
    kfn = pl.kernel(
        _sc_body,
        out_type=(
            jax.ShapeDtypeStruct((B // 2, 16), jnp.float32),
            jax.ShapeDtypeStruct((B // 16, 16), jnp.float32),
        ),
        mesh=mesh,
        compiler_params=pltpu.CompilerParams(needs_layout_passes=False),
        scratch_types=[
            pltpu.VMEM((N * BPW,), jnp.int32),
            pltpu.VMEM((N * BPW,), jnp.float32),
            pltpu.VMEM((N, 16), jnp.float32),
            pltpu.VMEM((16,), jnp.float32),
            pltpu.VMEM((BPW // 2, 16), jnp.int32),
            pltpu.VMEM((BPW,), jnp.float32),
            pltpu.VMEM((BPW // 2, 16), jnp.float32),
            [pltpu.SemaphoreType.DMA] * NG,
        ],
    )
    out, pz = kfn(xf, idx, maskb, act2, bsel_all)
    return out.reshape(B, 8), pz.reshape(1, B)
